# baseline (speedup 1.0000x reference)
import functools

import jax
import jax.numpy as jnp
from jax import lax
from jax.experimental import pallas as pl
from jax.experimental.pallas import tpu as pltpu
from jax.experimental.pallas import tpu_sc as plsc

N = 10000
E = 320000
D = 128
NC = 2
NS = 16
NW = NC * NS
B = 128
EPT = E // NW
NBS = 80
NBG = NBS + 2
EPAD = NBG * B
NPAD = 10240
NPT = NPAD // NS

_mesh = plsc.VectorSubcoreMesh(core_axis_name="c", subcore_axis_name="s")


@functools.partial(
    pl.kernel,
    mesh=_mesh,
    out_type=jax.ShapeDtypeStruct((NC, NPAD, D), jnp.float32),
    scratch_types=[
        pltpu.VMEM((NBS, B), jnp.int32),
        pltpu.VMEM((B, D), jnp.float32),
        pltpu.VMEM_SHARED((NPAD, D), jnp.float32),
        pltpu.SemaphoreType.DMA,
        pltpu.SemaphoreType.DMA,
    ],
)
def _sc_degree(dst_hbm, deg_out, dstall, buf, deg_sh, sema, semb):
    c = lax.axis_index("c")
    s = lax.axis_index("s")
    one16 = jnp.ones((16,), jnp.float32)
    zero16 = jnp.zeros((16,), jnp.float32)

    def fill_zero(i, carry):
        for k in range(D // 16):
            buf[i, pl.ds(k * 16, 16)] = zero16
        return carry

    lax.fori_loop(0, B, fill_zero, 0)
    for r in range(NPT // B):
        pltpu.sync_copy(buf, deg_sh.at[pl.ds(s * NPT + r * B, B)])

    def fill_one(i, carry):
        for k in range(D // 16):
            buf[i, pl.ds(k * 16, 16)] = one16
        return carry

    lax.fori_loop(0, B, fill_one, 0)
    pltpu.sync_copy(dst_hbm.at[c, s, pl.ds(0, NBS)], dstall)
    plsc.subcore_barrier()

    pltpu.async_copy(buf, deg_sh.at[dstall.at[0]], sema, add=True)
    pltpu.async_copy(buf, deg_sh.at[dstall.at[1]], semb, add=True)

    def blk2(k, carry):
        t0 = 2 * k
        pltpu.make_async_copy(buf, deg_sh.at[dstall.at[t0 - 2]], sema).wait()
        pltpu.async_copy(buf, deg_sh.at[dstall.at[t0]], sema, add=True)
        pltpu.make_async_copy(buf, deg_sh.at[dstall.at[t0 - 1]], semb).wait()
        pltpu.async_copy(buf, deg_sh.at[dstall.at[t0 + 1]], semb, add=True)
        return carry

    lax.fori_loop(1, NBS // 2, blk2, 0)
    pltpu.make_async_copy(buf, deg_sh.at[dstall.at[NBS - 2]], sema).wait()
    pltpu.make_async_copy(buf, deg_sh.at[dstall.at[NBS - 1]], semb).wait()

    plsc.subcore_barrier()
    pltpu.sync_copy(deg_sh.at[pl.ds(s * NPT, NPT)],
                    deg_out.at[c, pl.ds(s * NPT, NPT)])


@functools.partial(
    pl.kernel,
    mesh=_mesh,
    out_type=jax.ShapeDtypeStruct((NC, NPAD, D), jnp.float32),
    scratch_types=[
        pltpu.VMEM((NBG, B), jnp.int32),
        pltpu.VMEM((B,), jnp.int32),
        pltpu.VMEM((B,), jnp.int32),
        pltpu.VMEM((B, D), jnp.float32),
        pltpu.VMEM((B, D), jnp.float32),
        pltpu.VMEM_SHARED((NPAD, D), jnp.float32),
        pltpu.SemaphoreType.DMA,
        pltpu.SemaphoreType.DMA,
        pltpu.SemaphoreType.DMA,
        pltpu.SemaphoreType.DMA,
    ],
)
def _sc_aggregate(ht_hbm, src_hbm, dst_hbm, acc_out, srcall, didx0, didx1,
                  rows0, rows1, acc_sh, semg0, semg1, semd0, semd1):
    c = lax.axis_index("c")
    s = lax.axis_index("s")
    zero16 = jnp.zeros((16,), jnp.float32)

    def zrow(i, carry):
        for k in range(D // 16):
            rows0[i, pl.ds(k * 16, 16)] = zero16
        return carry

    lax.fori_loop(0, B, zrow, 0)
    for r in range(NPT // B):
        pltpu.sync_copy(rows0, acc_sh.at[pl.ds(s * NPT + r * B, B)])
    pltpu.sync_copy(src_hbm.at[c, s], srcall)
    plsc.subcore_barrier()

    pltpu.async_copy(dst_hbm.at[c, s, 0], didx0, semd0)
    pltpu.async_copy(ht_hbm.at[srcall.at[0]], rows0, semg0)
    pltpu.async_copy(ht_hbm.at[srcall.at[1]], rows1, semg1)

    def blk2(k, carry):
        t0 = 2 * k
        t1 = t0 + 1
        pltpu.async_copy(dst_hbm.at[c, s, t1], didx1, semd1)
        pltpu.make_async_copy(ht_hbm.at[srcall.at[t0]], rows0, semg0).wait()
        pltpu.make_async_copy(dst_hbm.at[c, s, t0], didx0, semd0).wait()
        pltpu.sync_copy(rows0, acc_sh.at[didx0], add=True)
        pltpu.async_copy(ht_hbm.at[srcall.at[t0 + 2]], rows0, semg0)
        pltpu.async_copy(dst_hbm.at[c, s, t1 + 1], didx0, semd0)
        pltpu.make_async_copy(ht_hbm.at[srcall.at[t1]], rows1, semg1).wait()
        pltpu.make_async_copy(dst_hbm.at[c, s, t1], didx1, semd1).wait()
        pltpu.sync_copy(rows1, acc_sh.at[didx1], add=True)
        pltpu.async_copy(ht_hbm.at[srcall.at[t1 + 2]], rows1, semg1)
        return carry

    lax.fori_loop(0, NBS // 2, blk2, 0)

    pltpu.make_async_copy(ht_hbm.at[srcall.at[NBS]], rows0, semg0).wait()
    pltpu.make_async_copy(ht_hbm.at[srcall.at[NBS + 1]], rows1, semg1).wait()
    pltpu.make_async_copy(dst_hbm.at[c, s, NBS], didx0, semd0).wait()

    plsc.subcore_barrier()
    pltpu.sync_copy(acc_sh.at[pl.ds(s * NPT, NPT)],
                    acc_out.at[c, pl.ds(s * NPT, NPT)])


M_BLK = 1000
GRID = N // M_BLK


def _dinv_block(deg):
    return lax.rsqrt(deg[0, :, 0:1] + deg[1, :, 0:1] + 1.0)


def _tc_mm(x_ref, w_ref, out_ref):
    out_ref[...] = jnp.dot(x_ref[...], w_ref[...],
                           preferred_element_type=jnp.float32)


def _tc_scale(deg_ref, h_ref, out_ref):
    out_ref[...] = h_ref[...] * _dinv_block(deg_ref[...])


def _tc_layer_mm(deg_ref, acc_ref, ht_ref, b_ref, w_ref, out_ref):
    dinv = _dinv_block(deg_ref[...])
    a = acc_ref[...]
    pre = dinv * (a[0] + a[1] + ht_ref[...]) + b_ref[...]
    z = jnp.maximum(pre, 0.0)
    h = jnp.dot(z, w_ref[...], preferred_element_type=jnp.float32)
    out_ref[...] = h * dinv


def _tc_final(deg_ref, acc_ref, ht_ref, b_ref, out_ref):
    dinv = _dinv_block(deg_ref[...])
    a = acc_ref[...]
    out_ref[...] = dinv * (a[0] + a[1] + ht_ref[...]) + b_ref[...]


def _acc_spec():
    return pl.BlockSpec((2, M_BLK, D), lambda i: (0, i, 0))


def _row_spec():
    return pl.BlockSpec((M_BLK, D), lambda i: (i, 0))


def _full_spec(r):
    return pl.BlockSpec((r, D), lambda i: (0, 0))


def _pad_edges(idx, pad_vals):
    per_tile = idx.reshape(NW, EPT)
    pads = jnp.broadcast_to(pad_vals[None, :], (NW, EPAD - EPT))
    return jnp.concatenate([per_tile, pads], axis=1).reshape(NC, NS, NBG, B)


def kernel(x, edge_index, W1, b1, W2, b2):
    ei = edge_index.astype(jnp.int32)
    npad = EPAD - EPT
    dst_pad = N + (jnp.arange(npad, dtype=jnp.int32) % (NPAD - N))
    src_pad = (jnp.arange(npad, dtype=jnp.int32) * 41) % N
    src = _pad_edges(ei[0], src_pad)
    dst = _pad_edges(ei[1], dst_pad)
    b1r = b1.reshape(1, D)
    b2r = b2.reshape(1, D)

    degT = _sc_degree(dst)

    h1 = pl.pallas_call(
        _tc_mm,
        grid=(GRID,),
        in_specs=[_row_spec(), _full_spec(D)],
        out_specs=_row_spec(),
        out_shape=jax.ShapeDtypeStruct((N, D), jnp.float32),
    )(x, W1)

    ht1 = pl.pallas_call(
        _tc_scale,
        grid=(GRID,),
        in_specs=[_acc_spec(), _row_spec()],
        out_specs=_row_spec(),
        out_shape=jax.ShapeDtypeStruct((N, D), jnp.float32),
    )(degT, h1)

    acc1 = _sc_aggregate(ht1, src, dst)

    ht2 = pl.pallas_call(
        _tc_layer_mm,
        grid=(GRID,),
        in_specs=[_acc_spec(), _acc_spec(), _row_spec(), _full_spec(1),
                  _full_spec(D)],
        out_specs=_row_spec(),
        out_shape=jax.ShapeDtypeStruct((N, D), jnp.float32),
    )(degT, acc1, ht1, b1r, W2)

    acc2 = _sc_aggregate(ht2, src, dst)

    out = pl.pallas_call(
        _tc_final,
        grid=(GRID,),
        in_specs=[_acc_spec(), _acc_spec(), _row_spec(), _full_spec(1)],
        out_specs=_row_spec(),
        out_shape=jax.ShapeDtypeStruct((N, D), jnp.float32),
    )(degT, acc2, ht2, b2r)

    return out


# speedup vs baseline: 1.0036x; 1.0036x over previous
import functools

import jax
import jax.numpy as jnp
from jax import lax
from jax.experimental import pallas as pl
from jax.experimental.pallas import tpu as pltpu
from jax.experimental.pallas import tpu_sc as plsc

N = 10000
E = 320000
D = 128
NC = 2
NS = 16
NW = NC * NS
B = 128
EPT = E // NW
NBS = 80
NBG = NBS + 2
EPAD = NBG * B
NPAD = 10240
NPT = NPAD // NS

_mesh = plsc.VectorSubcoreMesh(core_axis_name="c", subcore_axis_name="s")


@functools.partial(
    pl.kernel,
    mesh=_mesh,
    out_type=jax.ShapeDtypeStruct((NC, NPAD, D), jnp.float32),
    scratch_types=[
        pltpu.VMEM((NBS, B), jnp.int32),
        pltpu.VMEM((B, D), jnp.float32),
        pltpu.VMEM_SHARED((NPAD, D), jnp.float32),
        pltpu.SemaphoreType.DMA,
        pltpu.SemaphoreType.DMA,
    ],
)
def _sc_degree(dst_hbm, deg_out, dstall, buf, deg_sh, sema, semb):
    c = lax.axis_index("c")
    s = lax.axis_index("s")
    one16 = jnp.ones((16,), jnp.float32)
    zero16 = jnp.zeros((16,), jnp.float32)

    def fill_zero(i, carry):
        for k in range(D // 16):
            buf[i, pl.ds(k * 16, 16)] = zero16
        return carry

    lax.fori_loop(0, B, fill_zero, 0)
    for r in range(NPT // B):
        pltpu.sync_copy(buf, deg_sh.at[pl.ds(s * NPT + r * B, B)])

    def fill_one(i, carry):
        for k in range(D // 16):
            buf[i, pl.ds(k * 16, 16)] = one16
        return carry

    lax.fori_loop(0, B, fill_one, 0)
    pltpu.sync_copy(dst_hbm.at[c, s, pl.ds(0, NBS)], dstall)
    plsc.subcore_barrier()

    pltpu.async_copy(buf, deg_sh.at[dstall.at[0]], sema, add=True)
    pltpu.async_copy(buf, deg_sh.at[dstall.at[1]], semb, add=True)

    def blk2(k, carry):
        t0 = 2 * k
        pltpu.make_async_copy(buf, deg_sh.at[dstall.at[t0 - 2]], sema).wait()
        pltpu.async_copy(buf, deg_sh.at[dstall.at[t0]], sema, add=True)
        pltpu.make_async_copy(buf, deg_sh.at[dstall.at[t0 - 1]], semb).wait()
        pltpu.async_copy(buf, deg_sh.at[dstall.at[t0 + 1]], semb, add=True)
        return carry

    lax.fori_loop(1, NBS // 2, blk2, 0)
    pltpu.make_async_copy(buf, deg_sh.at[dstall.at[NBS - 2]], sema).wait()
    pltpu.make_async_copy(buf, deg_sh.at[dstall.at[NBS - 1]], semb).wait()

    plsc.subcore_barrier()
    pltpu.sync_copy(deg_sh.at[pl.ds(s * NPT, NPT)],
                    deg_out.at[c, pl.ds(s * NPT, NPT)])


@functools.partial(
    pl.kernel,
    mesh=_mesh,
    out_type=jax.ShapeDtypeStruct((NC, NPAD, D), jnp.float32),
    scratch_types=[
        pltpu.VMEM((NBG, B), jnp.int32),
        pltpu.VMEM((B,), jnp.int32),
        pltpu.VMEM((B,), jnp.int32),
        pltpu.VMEM((B, D), jnp.float32),
        pltpu.VMEM((B, D), jnp.float32),
        pltpu.VMEM_SHARED((NPAD, D), jnp.float32),
        pltpu.SemaphoreType.DMA,
        pltpu.SemaphoreType.DMA,
        pltpu.SemaphoreType.DMA,
        pltpu.SemaphoreType.DMA,
    ],
)
def _sc_aggregate(ht_hbm, src_hbm, dst_hbm, acc_out, srcall, didx0, didx1,
                  rows0, rows1, acc_sh, semg0, semg1, semd0, semd1):
    c = lax.axis_index("c")
    s = lax.axis_index("s")
    zero16 = jnp.zeros((16,), jnp.float32)

    def zrow(i, carry):
        for k in range(D // 16):
            rows0[i, pl.ds(k * 16, 16)] = zero16
        return carry

    lax.fori_loop(0, B, zrow, 0)
    for r in range(NPT // B):
        pltpu.sync_copy(rows0, acc_sh.at[pl.ds(s * NPT + r * B, B)])
    pltpu.sync_copy(src_hbm.at[c, s], srcall)
    plsc.subcore_barrier()

    pltpu.async_copy(dst_hbm.at[c, s, 0], didx0, semd0)
    pltpu.async_copy(ht_hbm.at[srcall.at[0]], rows0, semg0)
    pltpu.async_copy(ht_hbm.at[srcall.at[1]], rows1, semg1)

    def blk2(k, carry):
        t0 = 2 * k
        t1 = t0 + 1
        pltpu.async_copy(dst_hbm.at[c, s, t1], didx1, semd1)
        pltpu.make_async_copy(ht_hbm.at[srcall.at[t0]], rows0, semg0).wait()
        pltpu.make_async_copy(dst_hbm.at[c, s, t0], didx0, semd0).wait()
        pltpu.sync_copy(rows0, acc_sh.at[didx0], add=True)
        pltpu.async_copy(ht_hbm.at[srcall.at[t0 + 2]], rows0, semg0)
        pltpu.async_copy(dst_hbm.at[c, s, t1 + 1], didx0, semd0)
        pltpu.make_async_copy(ht_hbm.at[srcall.at[t1]], rows1, semg1).wait()
        pltpu.make_async_copy(dst_hbm.at[c, s, t1], didx1, semd1).wait()
        pltpu.sync_copy(rows1, acc_sh.at[didx1], add=True)
        pltpu.async_copy(ht_hbm.at[srcall.at[t1 + 2]], rows1, semg1)
        return carry

    lax.fori_loop(0, NBS // 2, blk2, 0)

    pltpu.make_async_copy(ht_hbm.at[srcall.at[NBS]], rows0, semg0).wait()
    pltpu.make_async_copy(ht_hbm.at[srcall.at[NBS + 1]], rows1, semg1).wait()
    pltpu.make_async_copy(dst_hbm.at[c, s, NBS], didx0, semd0).wait()

    plsc.subcore_barrier()
    pltpu.sync_copy(acc_sh.at[pl.ds(s * NPT, NPT)],
                    acc_out.at[c, pl.ds(s * NPT, NPT)])


M_BLK = 1000
GRID = N // M_BLK


def _dinv_block(deg):
    return lax.rsqrt(deg[0, :, 0:1] + deg[1, :, 0:1] + 1.0)


def _tc_scale_mm(deg_ref, x_ref, w_ref, out_ref):
    dinv = _dinv_block(deg_ref[...])
    h = jnp.dot(x_ref[...], w_ref[...], preferred_element_type=jnp.float32)
    out_ref[...] = h * dinv


def _tc_layer_mm(deg_ref, acc_ref, ht_ref, b_ref, w_ref, out_ref):
    dinv = _dinv_block(deg_ref[...])
    a = acc_ref[...]
    pre = dinv * (a[0] + a[1] + ht_ref[...]) + b_ref[...]
    z = jnp.maximum(pre, 0.0)
    h = jnp.dot(z, w_ref[...], preferred_element_type=jnp.float32)
    out_ref[...] = h * dinv


def _tc_final(deg_ref, acc_ref, ht_ref, b_ref, out_ref):
    dinv = _dinv_block(deg_ref[...])
    a = acc_ref[...]
    out_ref[...] = dinv * (a[0] + a[1] + ht_ref[...]) + b_ref[...]


def _acc_spec():
    return pl.BlockSpec((2, M_BLK, D), lambda i: (0, i, 0))


def _row_spec():
    return pl.BlockSpec((M_BLK, D), lambda i: (i, 0))


def _full_spec(r):
    return pl.BlockSpec((r, D), lambda i: (0, 0))


def _pad_edges(idx, pad_vals):
    per_tile = idx.reshape(NW, EPT)
    pads = jnp.broadcast_to(pad_vals[None, :], (NW, EPAD - EPT))
    return jnp.concatenate([per_tile, pads], axis=1).reshape(NC, NS, NBG, B)


def kernel(x, edge_index, W1, b1, W2, b2):
    ei = edge_index.astype(jnp.int32)
    npad = EPAD - EPT
    dst_pad = N + (jnp.arange(npad, dtype=jnp.int32) % (NPAD - N))
    src_pad = (jnp.arange(npad, dtype=jnp.int32) * 41) % N
    src = _pad_edges(ei[0], src_pad)
    dst = _pad_edges(ei[1], dst_pad)
    b1r = b1.reshape(1, D)
    b2r = b2.reshape(1, D)

    degT = _sc_degree(dst)

    ht1 = pl.pallas_call(
        _tc_scale_mm,
        grid=(GRID,),
        in_specs=[_acc_spec(), _row_spec(), _full_spec(D)],
        out_specs=_row_spec(),
        out_shape=jax.ShapeDtypeStruct((N, D), jnp.float32),
    )(degT, x, W1)

    acc1 = _sc_aggregate(ht1, src, dst)

    ht2 = pl.pallas_call(
        _tc_layer_mm,
        grid=(GRID,),
        in_specs=[_acc_spec(), _acc_spec(), _row_spec(), _full_spec(1),
                  _full_spec(D)],
        out_specs=_row_spec(),
        out_shape=jax.ShapeDtypeStruct((N, D), jnp.float32),
    )(degT, acc1, ht1, b1r, W2)

    acc2 = _sc_aggregate(ht2, src, dst)

    out = pl.pallas_call(
        _tc_final,
        grid=(GRID,),
        in_specs=[_acc_spec(), _acc_spec(), _row_spec(), _full_spec(1)],
        out_specs=_row_spec(),
        out_shape=jax.ShapeDtypeStruct((N, D), jnp.float32),
    )(degT, acc2, ht2, b2r)

    return out
